# baseline (speedup 1.0000x reference)
import functools

import jax
import jax.numpy as jnp
from jax import lax
from jax.experimental import pallas as pl
from jax.experimental.pallas import tpu as pltpu
from jax.experimental.pallas import tpu_sc as plsc

_NUM_BUCKETS = 1000
_EMBED_DIM = 128
_BATCH = 16384
_DIV = 100.0 + 1e-8


_NC = 2
_NS = 16
_L = 16
_NW = _NC * _NS
_B_SC = _BATCH // 2
_BPW = _B_SC // _NW
_CHUNK = 64
_NCHUNK = _BPW // _CHUNK
_ROWS = _NUM_BUCKETS
_RPT = 64


def _sc_body(x_hbm, table_hbm, out_hbm, x_v, idx_v, rows_v, table_sh, gsems):
    cid = lax.axis_index("c")
    sid = lax.axis_index("s")
    wid = sid * _NC + cid
    base = wid * _BPW

    start = pl.multiple_of(jnp.minimum(sid * _RPT, _ROWS - _RPT), 8)
    pltpu.sync_copy(
        table_hbm.at[pl.ds(start, _RPT)], table_sh.at[pl.ds(start, _RPT)]
    )

    pltpu.sync_copy(x_hbm.at[pl.ds(base, _BPW)], x_v)
    for j in range(_NCHUNK):
        for i in range(_CHUNK // _L):
            xv = x_v[pl.ds(j * _CHUNK + i * _L, _L)]
            y = (xv / jnp.float32(_DIV)) * jnp.float32(_NUM_BUCKETS)
            idx = jnp.clip(y.astype(jnp.int32), 0, _NUM_BUCKETS - 1)
            idx_v[j, pl.ds(i * _L, _L)] = idx

    plsc.subcore_barrier()

    gathers = [
        pltpu.async_copy(
            table_sh.at[idx_v.at[j]],
            rows_v.at[pl.ds(j * _CHUNK, _CHUNK)],
            gsems[j],
        )
        for j in range(_NCHUNK)
    ]
    for j in range(_NCHUNK):
        gathers[j].wait()
        pltpu.sync_copy(
            rows_v.at[pl.ds(j * _CHUNK, _CHUNK)],
            out_hbm.at[pl.ds(base + j * _CHUNK, _CHUNK)],
        )


_sc_lookup = functools.partial(
    pl.kernel,
    out_type=jax.ShapeDtypeStruct((_B_SC, _EMBED_DIM), jnp.float32),
    mesh=plsc.VectorSubcoreMesh(core_axis_name="c", subcore_axis_name="s"),
    scratch_types=[
        pltpu.VMEM((_BPW,), jnp.float32),
        pltpu.VMEM((_NCHUNK, _CHUNK), jnp.int32),
        pltpu.VMEM((_BPW, _EMBED_DIM), jnp.float32),
        pltpu.VMEM_SHARED((_ROWS, _EMBED_DIM), jnp.float32),
        [pltpu.SemaphoreType.DMA] * _NCHUNK,
    ],
)(_sc_body)


_PAD = 1024
_BLK = 1024
_B_TC = _BATCH - _B_SC


def _tc_body(x_ref, hi_ref, lo_ref, o_ref):
    xb = x_ref[...]
    y = (xb / jnp.float32(_DIV)) * jnp.float32(_NUM_BUCKETS)
    idx = jnp.clip(y.astype(jnp.int32), 0, _NUM_BUCKETS - 1)
    col = lax.broadcasted_iota(jnp.int32, (_BLK, _PAD), 1)
    oh = (idx == col).astype(jnp.bfloat16)
    acc = jnp.dot(oh, hi_ref[...], preferred_element_type=jnp.float32)
    acc = acc + jnp.dot(oh, lo_ref[...], preferred_element_type=jnp.float32)
    o_ref[...] = acc


_tc_lookup = pl.pallas_call(
    _tc_body,
    grid=(_B_TC // _BLK,),
    in_specs=[
        pl.BlockSpec((_BLK, 1), lambda i: (i, 0)),
        pl.BlockSpec((_PAD, _EMBED_DIM), lambda i: (0, 0)),
        pl.BlockSpec((_PAD, _EMBED_DIM), lambda i: (0, 0)),
    ],
    out_specs=pl.BlockSpec((_BLK, _EMBED_DIM), lambda i: (i, 0)),
    out_shape=jax.ShapeDtypeStruct((_B_TC, _EMBED_DIM), jnp.float32),
)


def kernel(x, table):
    sc_out = _sc_lookup(x[:_B_SC], table)
    tpad = jnp.zeros((_PAD, _EMBED_DIM), jnp.float32).at[:_NUM_BUCKETS].set(
        table[:_NUM_BUCKETS]
    )
    hi = tpad.astype(jnp.bfloat16)
    lo = (tpad - hi.astype(jnp.float32)).astype(jnp.bfloat16)
    tc_out = _tc_lookup(x[_B_SC:].reshape(_B_TC, 1), hi, lo)
    return jnp.concatenate([sc_out, tc_out], axis=0)


# speedup vs baseline: 1.6220x; 1.6220x over previous
import functools

import jax
import jax.numpy as jnp
from jax import lax
from jax.experimental import pallas as pl
from jax.experimental.pallas import tpu as pltpu
from jax.experimental.pallas import tpu_sc as plsc

_NUM_BUCKETS = 1000
_EMBED_DIM = 128
_BATCH = 16384
_DIV = 100.0 + 1e-8

_NC = 2
_NS = 16
_L = 16
_NW = _NC * _NS
_BPW = _BATCH // _NW
_CHUNK = 64
_NCHUNK = _BPW // _CHUNK
_ROWS = _NUM_BUCKETS
_RPT = 64


def _body(x_hbm, table_hbm, out_hbm, x_v, idx_v, rows_v, table_sh, gsems):
    cid = lax.axis_index("c")
    sid = lax.axis_index("s")
    wid = sid * _NC + cid
    base = wid * _BPW

    start = pl.multiple_of(jnp.minimum(sid * _RPT, _ROWS - _RPT), 8)
    pltpu.sync_copy(
        table_hbm.at[pl.ds(start, _RPT)], table_sh.at[pl.ds(start, _RPT)]
    )

    pltpu.sync_copy(x_hbm.at[pl.ds(base, _BPW)], x_v)
    for j in range(_NCHUNK):
        for i in range(_CHUNK // _L):
            xv = x_v[pl.ds(j * _CHUNK + i * _L, _L)]
            y = (xv / jnp.float32(_DIV)) * jnp.float32(_NUM_BUCKETS)
            idx = jnp.clip(y.astype(jnp.int32), 0, _NUM_BUCKETS - 1)
            idx_v[j, pl.ds(i * _L, _L)] = idx

    plsc.subcore_barrier()

    gathers = [
        pltpu.async_copy(
            table_sh.at[idx_v.at[j]],
            rows_v.at[pl.ds(j * _CHUNK, _CHUNK)],
            gsems[j],
        )
        for j in range(_NCHUNK)
    ]
    for j in range(_NCHUNK):
        gathers[j].wait()
        pltpu.sync_copy(
            rows_v.at[pl.ds(j * _CHUNK, _CHUNK)],
            out_hbm.at[pl.ds(base + j * _CHUNK, _CHUNK)],
        )


_sc_lookup = functools.partial(
    pl.kernel,
    out_type=jax.ShapeDtypeStruct((_BATCH, _EMBED_DIM), jnp.float32),
    mesh=plsc.VectorSubcoreMesh(core_axis_name="c", subcore_axis_name="s"),
    scratch_types=[
        pltpu.VMEM((_BPW,), jnp.float32),
        pltpu.VMEM((_NCHUNK, _CHUNK), jnp.int32),
        pltpu.VMEM((_BPW, _EMBED_DIM), jnp.float32),
        pltpu.VMEM_SHARED((_ROWS, _EMBED_DIM), jnp.float32),
        [pltpu.SemaphoreType.DMA] * _NCHUNK,
    ],
)(_body)


def kernel(x, table):
    return _sc_lookup(x, table)
